# baseline (speedup 1.0000x reference)
import jax
import jax.numpy as jnp
from jax.experimental import pallas as pl
from jax.experimental.pallas import tpu as pltpu

_IN_FEATURES = 28 * 28
_OUT_FEATURES = 5
_TAU = 2.0
_V_THRESHOLD = 0.8
_M_PAD = 8

_BM = 1024
_NS = 4


def _vpr_block_kernel(*refs):
    xt_refs = refs[:_NS]
    w_ref = refs[_NS]
    cur_refs = refs[_NS + 1:2 * _NS + 1]
    spk_refs = refs[2 * _NS + 1:]
    w = w_ref[...]
    for xt_ref, cur_ref, spk_ref in zip(xt_refs, cur_refs, spk_refs):
        xt = xt_ref[...]
        rows = [
            jnp.sum(xt * w[j, :, None], axis=0, keepdims=True)
            for j in range(_OUT_FEATURES)
        ]
        rows.append(jnp.zeros((_M_PAD - _OUT_FEATURES, xt.shape[1]), jnp.float32))
        cur = jnp.concatenate(rows, axis=0)
        cur_ref[...] = cur
        spk_ref[...] = (cur / _TAU >= _V_THRESHOLD).astype(jnp.float32)


@jax.jit
def kernel(xb, w_pad):
    b = xb.shape[0]
    xt = jnp.reshape(xb, (b, _IN_FEATURES)).astype(jnp.float32).T

    group = _BM * _NS
    ng = pl.cdiv(b, group)
    b_pad = ng * group
    if b_pad != b:
        xt = jnp.pad(xt, ((0, 0), (0, b_pad - b)))

    def x_map(k):
        return lambda i, k=k: (0, _NS * i + k)

    outs = pl.pallas_call(
        _vpr_block_kernel,
        out_shape=tuple(
            jax.ShapeDtypeStruct((_M_PAD, ng * _BM), jnp.float32)
            for _ in range(2 * _NS)
        ),
        grid=(ng,),
        in_specs=[
            pl.BlockSpec((_IN_FEATURES, _BM), x_map(k)) for k in range(_NS)
        ] + [pl.BlockSpec((_M_PAD, _IN_FEATURES), lambda i: (0, 0))],
        out_specs=tuple(
            pl.BlockSpec((_M_PAD, _BM), lambda i: (0, i)) for _ in range(2 * _NS)
        ),
        compiler_params=pltpu.CompilerParams(
            dimension_semantics=("arbitrary",)),
    )(*([xt] * _NS + [w_pad.astype(jnp.float32)]))

    def interleave(ts):
        stacked = jnp.stack([jnp.reshape(t, (_M_PAD, ng, _BM)) for t in ts],
                            axis=2)
        return jnp.reshape(stacked, (_M_PAD, b_pad))

    cur = interleave(outs[:_NS])[:_OUT_FEATURES, :b].T
    spikes = interleave(outs[_NS:])[:_OUT_FEATURES, :b].T
    return spikes, cur


# speedup vs baseline: 1.1601x; 1.1601x over previous
import jax
import jax.numpy as jnp
from jax.experimental import pallas as pl
from jax.experimental.pallas import tpu as pltpu

_IN_FEATURES = 28 * 28
_OUT_FEATURES = 5
_TAU = 2.0
_V_THRESHOLD = 0.8
_M_PAD = 8
_NS = 8


def _vpr_manual_kernel(x_hbm, w_ref, cur_ref, spk_ref, xbuf, sem):
    slab = xbuf.shape[2]
    w = w_ref[...]
    copies = [
        pltpu.make_async_copy(
            x_hbm.at[:, pl.ds(k * slab, slab)], xbuf.at[k], sem.at[k])
        for k in range(_NS)
    ]
    for c in copies:
        c.start()
    for k in range(_NS):
        copies[k].wait()
        xt = xbuf[k]
        rows = [
            jnp.sum(xt * w[j, :, None], axis=0, keepdims=True)
            for j in range(_OUT_FEATURES)
        ]
        rows.append(jnp.zeros((_M_PAD - _OUT_FEATURES, slab), jnp.float32))
        cur = jnp.concatenate(rows, axis=0)
        cur_ref[:, k * slab:(k + 1) * slab] = cur
        spk_ref[:, k * slab:(k + 1) * slab] = (
            cur / _TAU >= _V_THRESHOLD).astype(jnp.float32)


@jax.jit
def kernel(xb, w_pad):
    b = xb.shape[0]
    xt = jnp.reshape(xb, (b, _IN_FEATURES)).astype(jnp.float32).T

    unit = _NS * 128
    b_pad = pl.cdiv(b, unit) * unit
    if b_pad != b:
        xt = jnp.pad(xt, ((0, 0), (0, b_pad - b)))
    slab = b_pad // _NS

    cur_t, spk_t = pl.pallas_call(
        _vpr_manual_kernel,
        out_shape=(
            jax.ShapeDtypeStruct((_M_PAD, b_pad), jnp.float32),
            jax.ShapeDtypeStruct((_M_PAD, b_pad), jnp.float32),
        ),
        in_specs=[
            pl.BlockSpec(memory_space=pltpu.MemorySpace.HBM),
            pl.BlockSpec(memory_space=pltpu.MemorySpace.VMEM),
        ],
        out_specs=(
            pl.BlockSpec(memory_space=pltpu.MemorySpace.VMEM),
            pl.BlockSpec(memory_space=pltpu.MemorySpace.VMEM),
        ),
        scratch_shapes=[
            pltpu.VMEM((_NS, _IN_FEATURES, slab), jnp.float32),
            pltpu.SemaphoreType.DMA((_NS,)),
        ],
    )(xt, w_pad.astype(jnp.float32))

    cur = cur_t[:_OUT_FEATURES, :b].T
    spikes = spk_t[:_OUT_FEATURES, :b].T
    return spikes, cur


# speedup vs baseline: 1.2053x; 1.0389x over previous
import jax
import jax.numpy as jnp
from jax.experimental import pallas as pl
from jax.experimental.pallas import tpu as pltpu

_IN_FEATURES = 28 * 28
_OUT_FEATURES = 5
_TAU = 2.0
_V_THRESHOLD = 0.8
_M_PAD = 8

_BM = 1024


def _vpr_block_kernel(xt_ref, w_ref, out_ref):
    xt = xt_ref[...]
    w = w_ref[...]
    rows = [
        jnp.sum(xt * w[j, :, None], axis=0, keepdims=True)
        for j in range(_OUT_FEATURES)
    ]
    rows.append(jnp.zeros((_M_PAD - _OUT_FEATURES, xt.shape[1]), jnp.float32))
    cur = jnp.concatenate(rows, axis=0)
    spk = (cur / _TAU >= _V_THRESHOLD).astype(jnp.float32)
    out_ref[...] = jnp.concatenate([cur, spk], axis=0)


@jax.jit
def kernel(xb, w_pad):
    b = xb.shape[0]
    xt = jnp.reshape(xb, (b, _IN_FEATURES)).astype(jnp.float32).T

    bm = _BM if b >= _BM else max(128, b)
    nb = pl.cdiv(b, bm)
    b_pad = nb * bm
    if b_pad != b:
        xt = jnp.pad(xt, ((0, 0), (0, b_pad - b)))

    out = pl.pallas_call(
        _vpr_block_kernel,
        out_shape=jax.ShapeDtypeStruct((2 * _M_PAD, b_pad), jnp.float32),
        grid=(nb,),
        in_specs=[
            pl.BlockSpec((_IN_FEATURES, bm), lambda i: (0, i)),
            pl.BlockSpec((_M_PAD, _IN_FEATURES), lambda i: (0, 0)),
        ],
        out_specs=pl.BlockSpec((2 * _M_PAD, bm), lambda i: (0, i)),
        compiler_params=pltpu.CompilerParams(
            dimension_semantics=("parallel",)),
    )(xt, w_pad.astype(jnp.float32))

    cur = out[:_OUT_FEATURES, :b].T
    spikes = out[_M_PAD:_M_PAD + _OUT_FEATURES, :b].T
    return spikes, cur


# speedup vs baseline: 1.2200x; 1.0122x over previous
import jax
import jax.numpy as jnp
from jax.experimental import pallas as pl
from jax.experimental.pallas import tpu as pltpu

_IN_FEATURES = 28 * 28
_OUT_FEATURES = 5
_TAU = 2.0
_V_THRESHOLD = 0.8
_M_PAD = 8

_BM = 2048


def _vpr_block_kernel(xt_ref, w_ref, cur_ref, spk_ref):
    xt = xt_ref[...]
    w = w_ref[...]
    rows = [
        jnp.sum(xt * w[j, :, None], axis=0, keepdims=True)
        for j in range(_OUT_FEATURES)
    ]
    rows.append(jnp.zeros((_M_PAD - _OUT_FEATURES, xt.shape[1]), jnp.float32))
    cur = jnp.concatenate(rows, axis=0)
    cur_ref[...] = cur
    spk_ref[...] = (cur / _TAU >= _V_THRESHOLD).astype(jnp.float32)


@jax.jit
def kernel(xb, w_pad):
    b = xb.shape[0]
    xt = jnp.reshape(xb, (b, _IN_FEATURES)).astype(jnp.float32).T

    bm = _BM if b >= _BM else max(128, b)
    nb = pl.cdiv(b, bm)
    b_pad = nb * bm
    if b_pad != b:
        xt = jnp.pad(xt, ((0, 0), (0, b_pad - b)))

    cur_t, spk_t = pl.pallas_call(
        _vpr_block_kernel,
        out_shape=(
            jax.ShapeDtypeStruct((_M_PAD, b_pad), jnp.float32),
            jax.ShapeDtypeStruct((_M_PAD, b_pad), jnp.float32),
        ),
        grid=(nb,),
        in_specs=[
            pl.BlockSpec((_IN_FEATURES, bm), lambda i: (0, i)),
            pl.BlockSpec((_M_PAD, _IN_FEATURES), lambda i: (0, 0)),
        ],
        out_specs=(
            pl.BlockSpec((_M_PAD, bm), lambda i: (0, i)),
            pl.BlockSpec((_M_PAD, bm), lambda i: (0, i)),
        ),
        compiler_params=pltpu.CompilerParams(
            dimension_semantics=("parallel",)),
    )(xt, w_pad.astype(jnp.float32))

    cur = cur_t[:_OUT_FEATURES, :b].T
    spikes = spk_t[:_OUT_FEATURES, :b].T
    return spikes, cur
